# baseline (speedup 1.0000x reference)
import jax
import jax.numpy as jnp
from jax import lax
from jax.experimental import pallas as pl
from jax.experimental.pallas import tpu as pltpu
from jax.experimental.pallas import tpu_sc as plsc

D = 1024
NC, NS = 2, 16
NW = NC * NS


def _sc_body(emb_hbm, pos_hbm, tim_hbm, flg_hbm, pos_tab_hbm, tim_tab_hbm,
             out_hbm, idx, in_buf, tra, trb, sem_i, sem_g, sem_w):
    wid = lax.axis_index("c") * NS + lax.axis_index("s")
    s0 = wid * 128

    cf = pltpu.async_copy(flg_hbm.at[:, pl.ds(s0, 128)],
                          out_hbm.at[3 * D, wid], sem_i)

    def transpose_b(b):
        iot = lax.iota(jnp.int32, 16)

        @plsc.parallel_loop(0, 64, unroll=4)
        def body_c(c):
            cols_a = iot * 0 + c
            cols_b = cols_a + 64
            for j in range(8):
                rows = iot + 16 * j
                va = plsc.load_gather(in_buf, [rows, cols_a])
                vb = plsc.load_gather(in_buf, [rows, cols_b])
                tra[c, 0, b, pl.ds(16 * j, 16)] = va
                trb[c, 0, b, pl.ds(16 * j, 16)] = vb

    def do_section(read_b, out_base):
        def chunk(k, _):
            cbase = out_base + 128 * k
            for b in range(4):
                read_b(k, b).wait()
                transpose_b(b)
            wa = pltpu.async_copy(
                tra, out_hbm.at[pl.ds(cbase, 64), pl.ds(wid, 1)], sem_w)
            wb = pltpu.async_copy(
                trb, out_hbm.at[pl.ds(cbase + 64, 64), pl.ds(wid, 1)], sem_w)
            wa.wait()
            wb.wait()
            return ()
        lax.fori_loop(0, D // 128, chunk, ())

    def read_emb(k, b):
        return pltpu.async_copy(
            emb_hbm.at[b, pl.ds(s0, 128), pl.ds(128 * k, 128)],
            in_buf.at[:, pl.ds(0, 128)], sem_g)
    do_section(read_emb, 0)

    ci = pltpu.async_copy(pos_hbm.at[:, pl.ds(s0, 128)], idx, sem_i)
    ci.wait()

    def read_pos(k, b):
        return pltpu.async_copy(
            pos_tab_hbm.at[idx.at[b], pl.ds(128 * k, 128)],
            in_buf.at[:, pl.ds(0, 128)], sem_g)
    do_section(read_pos, D)

    ci = pltpu.async_copy(tim_hbm.at[:, pl.ds(s0, 128)], idx, sem_i)
    ci.wait()

    def read_tim(k, b):
        return pltpu.async_copy(
            tim_tab_hbm.at[idx.at[b], pl.ds(128 * k, 128)],
            in_buf.at[:, pl.ds(0, 128)], sem_g)
    do_section(read_tim, 2 * D)

    cf.wait()


def kernel(embeddings, positions, times, source_flags, pos_table, time_table):
    B, S, Dm = embeddings.shape
    pos = positions.astype(jnp.int32)
    tim = times.astype(jnp.int32)
    flg = source_flags.astype(jnp.float32)
    mesh = plsc.VectorSubcoreMesh(
        core_axis_name="c", subcore_axis_name="s",
        num_cores=NC, num_subcores=NS)
    out4 = pl.kernel(
        _sc_body,
        out_type=jax.ShapeDtypeStruct((3 * Dm + 1, S // 128, B, 128),
                                      jnp.float32),
        mesh=mesh,
        compiler_params=pltpu.CompilerParams(needs_layout_passes=False),
        scratch_types=[
            pltpu.VMEM((B, 128), jnp.int32),
            pltpu.VMEM((128, 129), jnp.float32),
            pltpu.VMEM((64, 1, B, 128), jnp.float32),
            pltpu.VMEM((64, 1, B, 128), jnp.float32),
            pltpu.SemaphoreType.DMA,
            pltpu.SemaphoreType.DMA,
            pltpu.SemaphoreType.DMA,
        ],
    )(embeddings, pos, tim, flg, pos_table, time_table)
    return out4.transpose(2, 1, 3, 0).reshape(B, S, 3 * Dm + 1)


# speedup vs baseline: 2.9691x; 2.9691x over previous
import jax
import jax.numpy as jnp
from jax import lax
from jax.experimental import pallas as pl
from jax.experimental.pallas import tpu as pltpu
from jax.experimental.pallas import tpu_sc as plsc

D = 1024
NC, NS = 2, 16
NW = NC * NS
CH = 8


def _sc_body(emb_hbm, pos_hbm, tim_hbm, flg_hbm, pos_tab_hbm, tim_tab_hbm,
             out_hbm, idx2, flg_v, bufs,
             sem_i, sem_f, sem_ga, sem_gb, sem_w):
    T = pos_hbm.shape[0]
    tpw = T // NW
    pos_idx = idx2.at[0]
    tim_idx = idx2.at[1]
    pos_a = bufs.at[0]
    tim_a = bufs.at[1]
    emb_a = bufs.at[2]
    pos_b = bufs.at[3]
    tim_b = bufs.at[4]
    emb_b = bufs.at[5]
    wid = lax.axis_index("c") * NS + lax.axis_index("s")
    base = wid * tpw

    ci0 = pltpu.async_copy(pos_hbm.at[pl.ds(base, tpw)], pos_idx, sem_i)
    ci1 = pltpu.async_copy(tim_hbm.at[pl.ds(base, tpw)], tim_idx, sem_i)
    ci2 = pltpu.async_copy(flg_hbm.at[pl.ds(base, tpw)], flg_v, sem_i)
    ci0.wait()
    ci1.wait()
    ci2.wait()
    cf = pltpu.async_copy(flg_v, out_hbm.at[pl.ds(base, tpw), pl.ds(3 * D, 1)],
                          sem_f)

    def gathers(tok, off, pos_buf, tim_buf, emb_buf, sem):
        return (
            pltpu.async_copy(
                pos_tab_hbm.at[pos_idx.at[pl.ds(off, CH)]], pos_buf, sem),
            pltpu.async_copy(
                tim_tab_hbm.at[tim_idx.at[pl.ds(off, CH)]], tim_buf, sem),
            pltpu.async_copy(emb_hbm.at[pl.ds(tok, CH)], emb_buf, sem),
        )

    def writes(tok, pos_buf, tim_buf, emb_buf):
        return (
            pltpu.async_copy(
                emb_buf, out_hbm.at[pl.ds(tok, CH), pl.ds(0, D)], sem_w),
            pltpu.async_copy(
                pos_buf, out_hbm.at[pl.ds(tok, CH), pl.ds(D, D)], sem_w),
            pltpu.async_copy(
                tim_buf, out_hbm.at[pl.ds(tok, CH), pl.ds(2 * D, D)], sem_w),
        )

    def pair(i, _):
        t0 = base + (2 * i) * CH
        t1 = t0 + CH
        o0 = (2 * i) * CH
        ga = gathers(t0, o0, pos_a, tim_a, emb_a, sem_ga)
        gb = gathers(t1, o0 + CH, pos_b, tim_b, emb_b, sem_gb)
        for h in ga:
            h.wait()
        wa = writes(t0, pos_a, tim_a, emb_a)
        for h in gb:
            h.wait()
        wb = writes(t1, pos_b, tim_b, emb_b)
        for h in wa:
            h.wait()
        for h in wb:
            h.wait()
        return ()

    lax.fori_loop(0, tpw // (2 * CH), pair, ())
    cf.wait()


def kernel(embeddings, positions, times, source_flags, pos_table, time_table):
    B, S, Dm = embeddings.shape
    T = B * S
    tpw = T // NW
    emb = embeddings.reshape(T, Dm)
    pos = positions.reshape(T).astype(jnp.int32)
    tim = times.reshape(T).astype(jnp.int32)
    flg = source_flags.reshape(T, 1).astype(jnp.float32)
    mesh = plsc.VectorSubcoreMesh(
        core_axis_name="c", subcore_axis_name="s",
        num_cores=NC, num_subcores=NS)
    out = pl.kernel(
        _sc_body,
        out_type=jax.ShapeDtypeStruct((T, 3 * Dm + 1), jnp.float32),
        mesh=mesh,
        scratch_types=[
            pltpu.VMEM((2, tpw), jnp.int32),
            pltpu.VMEM((tpw, 1), jnp.float32),
            pltpu.VMEM((6, CH, Dm), jnp.float32),
            pltpu.SemaphoreType.DMA,
            pltpu.SemaphoreType.DMA,
            pltpu.SemaphoreType.DMA,
            pltpu.SemaphoreType.DMA,
            pltpu.SemaphoreType.DMA,
        ],
    )(emb, pos, tim, flg, pos_table, time_table)
    return out.reshape(B, S, 3 * Dm + 1)
